# Initial kernel scaffold; baseline (speedup 1.0000x reference)
import jax
import jax.numpy as jnp
from jax.experimental import pallas as pl


def kernel(inputs, embeddings):
    raise NotImplementedError("write your pallas kernel here")


# speedup vs baseline: 2.7840x; 2.7840x over previous
import functools

import jax
import jax.numpy as jnp
from jax import lax
from jax.experimental import pallas as pl
from jax.experimental.pallas import tpu as pltpu
from jax.experimental.pallas import tpu_sc as plsc

VOCAB = 1000000
EMBED = 64
BATCH = 16384
HIST = 200
PAD_VALUE = 0

NC = 2
NS = 16
NW = NC * NS
BPW = BATCH // NW
RG = 128
G = BPW // RG
CV = EMBED // 16
NBUF = 2


def _sc_body(idx_hbm, table_hbm, out_hbm, idx_v, buf_v, acc_v, e0_v, a_v, b_v,
             sem0, sem1):
    c = lax.axis_index("c")
    s = lax.axis_index("s")
    wid = s * NC + c

    pltpu.sync_copy(table_hbm.at[pl.ds(0, 1)], e0_v)
    zero = jnp.zeros((16,), jnp.float32)
    sems = (sem0, sem1)

    def group(g, _):
        pltpu.sync_copy(idx_hbm.at[wid, g], idx_v)

        def zrow(j, _):
            for cc in range(CV):
                acc_v[j, pl.ds(cc * 16, 16)] = zero
            return 0
        lax.fori_loop(0, RG, zrow, 0, unroll=4)

        for b in range(NBUF):
            pltpu.async_copy(table_hbm.at[idx_v.at[b]], buf_v.at[b], sems[b])

        def consume(p, b):
            pltpu.make_async_copy(table_hbm.at[idx_v.at[p]], buf_v.at[b],
                                  sems[b]).wait()

            def row(j, _):
                for cc in range(CV):
                    x = buf_v[b, j, pl.ds(cc * 16, 16)]
                    plsc.addupdate(acc_v.at[j, pl.ds(cc * 16, 16)], x)
                return 0
            lax.fori_loop(0, RG, row, 0, unroll=4)

            @pl.when(p + NBUF < HIST)
            def _fire():
                pltpu.async_copy(table_hbm.at[idx_v.at[p + NBUF]],
                                 buf_v.at[b], sems[b])
            return 0

        def pos(p, _):
            for b in range(NBUF):
                consume(p * NBUF + b, b)
            return 0
        lax.fori_loop(0, HIST // NBUF, pos, 0)

        def count(p, cnt):
            out = []
            for cc in range(8):
                v = idx_v[p, pl.ds(cc * 16, 16)]
                out.append(cnt[cc] + jnp.where(v != PAD_VALUE, 1.0, 0.0))
            return tuple(out)
        cnt = lax.fori_loop(0, HIST, count,
                            tuple(zero for _ in range(8)), unroll=2)
        for cc in range(8):
            a = 1.0 / (cnt[cc] + 1e-8)
            b = (cnt[cc] - float(HIST)) * a
            a_v[pl.ds(cc * 16, 16)] = a
            b_v[pl.ds(cc * 16, 16)] = b

        e0 = [e0_v[0, pl.ds(cc * 16, 16)] for cc in range(CV)]

        def frow(j, _):
            ji = jnp.full((16,), 0, jnp.int32) + j
            asp = plsc.load_gather(a_v, [ji])
            bsp = plsc.load_gather(b_v, [ji])
            for cc in range(CV):
                x = acc_v[j, pl.ds(cc * 16, 16)]
                acc_v[j, pl.ds(cc * 16, 16)] = x * asp + e0[cc] * bsp
            return 0
        lax.fori_loop(0, RG, frow, 0, unroll=2)

        row0 = (wid * G + g) * RG
        pltpu.sync_copy(acc_v, out_hbm.at[pl.ds(row0, RG)])
        return 0

    lax.fori_loop(0, G, group, 0)


@jax.jit
def _run(idx_arranged, embeddings):
    mesh = plsc.VectorSubcoreMesh(core_axis_name="c", subcore_axis_name="s")
    fn = pl.kernel(
        _sc_body,
        out_type=jax.ShapeDtypeStruct((BATCH, EMBED), jnp.float32),
        mesh=mesh,
        scratch_types=[
            pltpu.VMEM((HIST, RG), jnp.int32),
            pltpu.VMEM((NBUF, RG, EMBED), jnp.float32),
            pltpu.VMEM((RG, EMBED), jnp.float32),
            pltpu.VMEM((1, EMBED), jnp.float32),
            pltpu.VMEM((RG,), jnp.float32),
            pltpu.VMEM((RG,), jnp.float32),
            pltpu.SemaphoreType.DMA,
            pltpu.SemaphoreType.DMA,
        ],
        compiler_params=pltpu.CompilerParams(use_tc_tiling_on_sc=False, needs_layout_passes=False),
    )
    return fn(idx_arranged, embeddings)


def kernel(inputs, embeddings):
    idx = inputs.astype(jnp.int32).reshape(NW, G, RG, HIST)
    idx = idx.transpose(0, 1, 3, 2)
    return _run(idx, embeddings)


# speedup vs baseline: 3.1417x; 1.1285x over previous
import functools

import jax
import jax.numpy as jnp
from jax import lax
from jax.experimental import pallas as pl
from jax.experimental.pallas import tpu as pltpu
from jax.experimental.pallas import tpu_sc as plsc

VOCAB = 1000000
EMBED = 64
BATCH = 16384
HIST = 200
PAD_VALUE = 0

NC = 2
NS = 16
NW = NC * NS
BPW = BATCH // NW
RG = 128
G = BPW // RG
CV = EMBED // 16
NBUF = 4


def _sc_body(idx_hbm, table_hbm, out_hbm, idx_v, buf_v, acc_v, e0_v, a_v, b_v,
             sem0, sem1, sem2, sem3):
    c = lax.axis_index("c")
    s = lax.axis_index("s")
    wid = s * NC + c

    pltpu.sync_copy(table_hbm.at[pl.ds(0, 1)], e0_v)
    zero = jnp.zeros((16,), jnp.float32)
    sems = (sem0, sem1, sem2, sem3)

    def group(g, _):
        pltpu.sync_copy(idx_hbm.at[wid, g], idx_v)

        @plsc.parallel_loop(0, RG, unroll=8)
        def _zrow(j):
            for cc in range(CV):
                acc_v[j, pl.ds(cc * 16, 16)] = zero

        for b in range(NBUF):
            pltpu.async_copy(table_hbm.at[idx_v.at[b]], buf_v.at[b], sems[b])

        def consume(p, b):
            pltpu.make_async_copy(table_hbm.at[idx_v.at[p]], buf_v.at[b],
                                  sems[b]).wait()

            @plsc.parallel_loop(0, RG, unroll=8)
            def _row(j):
                for cc in range(CV):
                    x = buf_v[b, j, pl.ds(cc * 16, 16)]
                    plsc.addupdate(acc_v.at[j, pl.ds(cc * 16, 16)], x)

            @pl.when(p + NBUF < HIST)
            def _fire():
                pltpu.async_copy(table_hbm.at[idx_v.at[p + NBUF]],
                                 buf_v.at[b], sems[b])
            return 0

        def pos(p, _):
            for b in range(NBUF):
                consume(p * NBUF + b, b)
            return 0
        lax.fori_loop(0, HIST // NBUF, pos, 0)

        def count(p, cnt):
            out = []
            for cc in range(8):
                v = idx_v[p, pl.ds(cc * 16, 16)]
                out.append(cnt[cc] + jnp.where(v != PAD_VALUE, 1.0, 0.0))
            return tuple(out)
        cnt = lax.fori_loop(0, HIST, count,
                            tuple(zero for _ in range(8)), unroll=4)
        for cc in range(8):
            a = 1.0 / (cnt[cc] + 1e-8)
            b = (cnt[cc] - float(HIST)) * a
            a_v[pl.ds(cc * 16, 16)] = a
            b_v[pl.ds(cc * 16, 16)] = b

        e0 = [e0_v[0, pl.ds(cc * 16, 16)] for cc in range(CV)]

        def frow(j, _):
            ji = jnp.full((16,), 0, jnp.int32) + j
            asp = plsc.load_gather(a_v, [ji])
            bsp = plsc.load_gather(b_v, [ji])
            for cc in range(CV):
                x = acc_v[j, pl.ds(cc * 16, 16)]
                acc_v[j, pl.ds(cc * 16, 16)] = x * asp + e0[cc] * bsp
            return 0
        lax.fori_loop(0, RG, frow, 0, unroll=2)

        row0 = (wid * G + g) * RG
        pltpu.sync_copy(acc_v, out_hbm.at[pl.ds(row0, RG)])
        return 0

    lax.fori_loop(0, G, group, 0)


@jax.jit
def _run(idx_arranged, embeddings):
    mesh = plsc.VectorSubcoreMesh(core_axis_name="c", subcore_axis_name="s")
    fn = pl.kernel(
        _sc_body,
        out_type=jax.ShapeDtypeStruct((BATCH, EMBED), jnp.float32),
        mesh=mesh,
        scratch_types=[
            pltpu.VMEM((HIST, RG), jnp.int32),
            pltpu.VMEM((NBUF, RG, EMBED), jnp.float32),
            pltpu.VMEM((RG, EMBED), jnp.float32),
            pltpu.VMEM((1, EMBED), jnp.float32),
            pltpu.VMEM((RG,), jnp.float32),
            pltpu.VMEM((RG,), jnp.float32),
            pltpu.SemaphoreType.DMA,
            pltpu.SemaphoreType.DMA,
            pltpu.SemaphoreType.DMA,
            pltpu.SemaphoreType.DMA,
        ],
        compiler_params=pltpu.CompilerParams(use_tc_tiling_on_sc=False, needs_layout_passes=False),
    )
    return fn(idx_arranged, embeddings)


def kernel(inputs, embeddings):
    idx = inputs.astype(jnp.int32).reshape(NW, G, RG, HIST)
    idx = idx.transpose(0, 1, 3, 2)
    return _run(idx, embeddings)
